# Initial kernel scaffold; baseline (speedup 1.0000x reference)
import jax
import jax.numpy as jnp
from jax.experimental import pallas as pl


def kernel(x, edge_index, edge_attr, ne_w1, ne_w2, ee_w1, ee_w2, rel_w1, rel_b1, rel_w2, rel_b2, obj_w1, obj_b1, obj_w2, obj_b2, de_w1, de_w2, latent_norm):
    raise NotImplementedError("write your pallas kernel here")


# speedup vs baseline: 2.7670x; 2.7670x over previous
import functools

import jax
import jax.numpy as jnp
from jax import lax
from jax.experimental import pallas as pl
from jax.experimental.pallas import tpu as pltpu
from jax.experimental.pallas import tpu_sc as plsc

N_NODES = 10000
HIDDEN = 40
OUT_DIM = 8
W = 48
L = 16
NC = 2
NS = 16
NW = NC * NS
CHUNK = 128
ROWS_PT = 632
ACC_ROWS = NS * ROWS_PT
ALPHA = 0.5
ALPHA_FCNN = 0.5
NEG = -1e30


def _node_stage(x_ref, w1_ref, w2_ref, wd_ref, ws_ref, h_ref, hd_ref, hs_ref):
    h1 = jnp.maximum(jnp.dot(x_ref[...], w1_ref[...], preferred_element_type=jnp.float32), 0.0)
    h = jnp.dot(h1, w2_ref[...], preferred_element_type=jnp.float32)
    h_ref[...] = h
    hd_ref[...] = jnp.dot(h, wd_ref[...], preferred_element_type=jnp.float32)
    hs_ref[...] = jnp.dot(h, ws_ref[...], preferred_element_type=jnp.float32)


def _edge_stage(ea_ref, w1_ref, w2_ref, wc_ref, brow_ref, ep_ref):
    t = jnp.maximum(jnp.dot(ea_ref[...], w1_ref[...], preferred_element_type=jnp.float32), 0.0)
    e = jnp.dot(t, w2_ref[...], preferred_element_type=jnp.float32)
    ep_ref[...] = jnp.dot(e, wc_ref[...], preferred_element_type=jnp.float32) + brow_ref[...]


def _out_stage(h_ref, p0_ref, p1_ref, xfc_ref, rpad_ref, o1h_ref, o1a_ref,
               ob1_ref, ow2_ref, ob2_ref, dw1_ref, dw2_ref, ln_ref, out_ref):
    p = p0_ref[...] + p1_ref[...]
    aggr = jnp.dot(p, rpad_ref[...], preferred_element_type=jnp.float32)
    h = h_ref[...]
    t = jnp.maximum(
        jnp.dot(h, o1h_ref[...], preferred_element_type=jnp.float32)
        + jnp.dot(aggr, o1a_ref[...], preferred_element_type=jnp.float32)
        + ob1_ref[...], 0.0)
    dx = jnp.dot(t, ow2_ref[...], preferred_element_type=jnp.float32) + ob2_ref[...]
    h2 = ALPHA * h + (1.0 - ALPHA) * dx
    d2 = jnp.dot(jnp.maximum(jnp.dot(h2, dw1_ref[...], preferred_element_type=jnp.float32), 0.0),
                 dw2_ref[...], preferred_element_type=jnp.float32)
    out_ref[...] = (ALPHA_FCNN * xfc_ref[...] + (1.0 - ALPHA_FCNN) * d2) * ln_ref[...]


def _make_sc_edge(n_chunks):
    edges_per_worker = n_chunks * CHUNK
    mesh = plsc.VectorSubcoreMesh(
        core_axis_name="c", subcore_axis_name="s", num_cores=NC, num_subcores=NS)

    @functools.partial(
        pl.kernel,
        mesh=mesh,
        compiler_params=pltpu.CompilerParams(use_tc_tiling_on_sc=False),
        out_type=jax.ShapeDtypeStruct((NC, ACC_ROWS, W), jnp.float32),
        scratch_types=[
            pltpu.VMEM((CHUNK,), jnp.int32),
            pltpu.VMEM((CHUNK,), jnp.int32),
            pltpu.VMEM((CHUNK, W), jnp.float32),
            pltpu.VMEM((CHUNK, W), jnp.float32),
            pltpu.VMEM((CHUNK, W), jnp.float32),
            pltpu.VMEM_SHARED((ACC_ROWS, W), jnp.float32),
            pltpu.SemaphoreType.DMA,
            pltpu.SemaphoreType.DMA,
        ],
    )
    def sc_edge(dst_hbm, src_hbm, ep_hbm, hd_hbm, hs_hbm, zero_hbm, out_hbm,
                dix, six, av, bv, cv, acc, sem_a, sem_b):
        cid = lax.axis_index("c")
        sid = lax.axis_index("s")
        pltpu.sync_copy(zero_hbm, acc.at[pl.ds(sid * ROWS_PT, ROWS_PT)])
        plsc.subcore_barrier()
        base = (cid * NS + sid) * edges_per_worker

        def body(g, carry):
            eb = base + g * CHUNK
            pltpu.sync_copy(dst_hbm.at[pl.ds(eb, CHUNK)], dix)
            pltpu.sync_copy(src_hbm.at[pl.ds(eb, CHUNK)], six)
            cpa = pltpu.async_copy(hd_hbm.at[dix], av, sem_a)
            cpb = pltpu.async_copy(hs_hbm.at[six], bv, sem_b)
            pltpu.sync_copy(ep_hbm.at[pl.ds(eb, CHUNK)], cv)
            cpa.wait()
            cpb.wait()

            def inner(i, c2):
                for j in range(W // L):
                    sl = pl.ds(j * L, L)
                    cv[i, sl] = jnp.maximum(av[i, sl] + bv[i, sl] + cv[i, sl], 0.0)
                return c2

            lax.fori_loop(0, CHUNK, inner, 0)
            pltpu.sync_copy(cv, acc.at[dix], add=True)
            return carry

        lax.fori_loop(0, n_chunks, body, 0)
        plsc.subcore_barrier()
        pltpu.sync_copy(acc.at[pl.ds(sid * ROWS_PT, ROWS_PT)],
                        out_hbm.at[cid, pl.ds(sid * ROWS_PT, ROWS_PT)])

    return sc_edge


@jax.jit
def kernel(x, edge_index, edge_attr, ne_w1, ne_w2, ee_w1, ee_w2, rel_w1,
           rel_b1, rel_w2, rel_b2, obj_w1, obj_b1, obj_w2, obj_b2, de_w1,
           de_w2, latent_norm):
    f32 = jnp.float32
    n = x.shape[0]
    e_cnt = edge_attr.shape[0]

    def pad48(w):
        return jnp.concatenate([w, jnp.zeros((w.shape[0], W - HIDDEN), w.dtype)], axis=1)

    wd = pad48(rel_w1[0:HIDDEN])
    ws = pad48(rel_w1[HIDDEN:2 * HIDDEN])
    wc = pad48(rel_w1[2 * HIDDEN:3 * HIDDEN])
    brow = jnp.concatenate(
        [rel_b1, jnp.ones((1,), f32), jnp.zeros((W - HIDDEN - 1,), f32)]).reshape(1, W)
    rpad = jnp.concatenate(
        [rel_w2, rel_b2.reshape(1, HIDDEN), jnp.zeros((W - HIDDEN - 1, HIDDEN), f32)], axis=0)

    h, hd, hs = pl.pallas_call(
        _node_stage,
        out_shape=[
            jax.ShapeDtypeStruct((n, HIDDEN), f32),
            jax.ShapeDtypeStruct((n, W), f32),
            jax.ShapeDtypeStruct((n, W), f32),
        ],
    )(x, ne_w1, ne_w2, wd, ws)

    eb = 20000
    ep = pl.pallas_call(
        _edge_stage,
        grid=(e_cnt // eb,),
        in_specs=[
            pl.BlockSpec((eb, edge_attr.shape[1]), lambda i: (i, 0)),
            pl.BlockSpec(ee_w1.shape, lambda i: (0, 0)),
            pl.BlockSpec(ee_w2.shape, lambda i: (0, 0)),
            pl.BlockSpec((HIDDEN, W), lambda i: (0, 0)),
            pl.BlockSpec((1, W), lambda i: (0, 0)),
        ],
        out_specs=pl.BlockSpec((eb, W), lambda i: (i, 0)),
        out_shape=jax.ShapeDtypeStruct((e_cnt, W), f32),
    )(edge_attr, ee_w1, ee_w2, wc, brow)

    group = NW * CHUNK
    e_pad = ((e_cnt + group - 1) // group) * group
    n_chunks = e_pad // (NW * CHUNK)
    pad_amt = e_pad - e_cnt
    ep_p = jnp.concatenate([ep, jnp.full((pad_amt, W), NEG, f32)], axis=0)
    dst_p = jnp.concatenate([edge_index[1], jnp.zeros((pad_amt,), edge_index.dtype)])
    src_p = jnp.concatenate([edge_index[0], jnp.zeros((pad_amt,), edge_index.dtype)])
    zeros_tile = jnp.zeros((ROWS_PT, W), f32)

    parts = _make_sc_edge(n_chunks)(dst_p, src_p, ep_p, hd, hs, zeros_tile)

    out = pl.pallas_call(
        _out_stage,
        out_shape=jax.ShapeDtypeStruct((n, OUT_DIM), f32),
    )(h, parts[0, :n], parts[1, :n], x[:, :OUT_DIM], rpad,
      obj_w1[:HIDDEN], obj_w1[HIDDEN:], obj_b1.reshape(1, HIDDEN),
      obj_w2, obj_b2.reshape(1, HIDDEN), de_w1, de_w2, latent_norm.reshape(1, 1))
    return out
